# Initial kernel scaffold; baseline (speedup 1.0000x reference)
import jax
import jax.numpy as jnp
from jax.experimental import pallas as pl


def kernel(x, token_emb, pos_emb):
    raise NotImplementedError("write your pallas kernel here")


# speedup vs baseline: 1.1468x; 1.1468x over previous
import functools

import jax
import jax.numpy as jnp
from jax import lax
from jax.experimental import pallas as pl
from jax.experimental.pallas import tpu as pltpu
from jax.experimental.pallas import tpu_sc as plsc

B, S, D, V = 4, 2048, 768, 100000
NC, NS = 2, 16
NW = NC * NS
CHUNK = S // NW
LANES = 16


def _build():
    mesh = plsc.VectorSubcoreMesh(core_axis_name="c", subcore_axis_name="s")

    @functools.partial(
        pl.kernel,
        mesh=mesh,
        out_type=jax.ShapeDtypeStruct((B, S, D), jnp.float32),
        scratch_types=[
            pltpu.VMEM((B, CHUNK), jnp.int32),
            pltpu.VMEM((CHUNK, D), jnp.float32),
            pltpu.VMEM((CHUNK, D), jnp.float32),
            pltpu.SemaphoreType.DMA,
        ],
    )
    def emb_kernel(x_hbm, tok_hbm, pos_hbm, out_hbm, idx_v, pos_v, row_v, sem):
        wid = lax.axis_index("s") * NC + lax.axis_index("c")
        base = wid * CHUNK

        pltpu.sync_copy(pos_hbm.at[pl.ds(base, CHUNK)], pos_v)
        for b in range(B):
            pltpu.sync_copy(x_hbm.at[b, pl.ds(base, CHUNK)], idx_v.at[b])

        for b in range(B):
            pltpu.async_copy(tok_hbm.at[idx_v.at[b]], row_v, sem).wait()

            def add_row(r, _):
                for j in range(D // LANES):
                    sl = pl.ds(j * LANES, LANES)
                    row_v[r, sl] = row_v[r, sl] + pos_v[r, sl]
                return 0

            lax.fori_loop(0, CHUNK, add_row, 0)
            pltpu.sync_copy(row_v, out_hbm.at[b, pl.ds(base, CHUNK)])

    return emb_kernel


_emb = _build()


def kernel(x, token_emb, pos_emb):
    return _emb(x.astype(jnp.int32), token_emb, pos_emb)
